# baseline (speedup 1.0000x reference)
import jax
import jax.numpy as jnp
from jax import lax
from jax.experimental import pallas as pl
from jax.experimental.pallas import tpu as pltpu
from jax.experimental.pallas import tpu_sc as plsc

F = 26
NV = 100000
B = 16384
NC = 2
NS = 16
NW = NC * NS
BPW = B // NW
CHUNK = BPW * F
SPF = BPW // 16
RG = BPW // 16


def _idx_body(xt_hbm, idx_hbm, x_v, idx_v, sem):
    c = lax.axis_index("c")
    s = lax.axis_index("s")
    wid = s * NC + c
    base = wid * BPW

    copies = [
        pltpu.make_async_copy(
            xt_hbm.at[pl.ds(f * B + base, BPW)],
            x_v.at[pl.ds(f * BPW, BPW)],
            sem,
        )
        for f in range(F)
    ]
    for cp in copies:
        cp.start()

    for f in range(F):
        copies[f].wait()

        def add_off(i, carry, f=f):
            j = f * SPF + i
            idx_v[pl.ds(j * 16, 16)] = x_v[pl.ds(j * 16, 16)] + (f * NV)
            return carry

        lax.fori_loop(0, SPF, add_off, 0)

    pltpu.sync_copy(idx_v, idx_hbm.at[pl.ds(wid * CHUNK, CHUNK)])


def _gather_body(idx_hbm, table_hbm, bias_hbm, out_hbm, idx_v, g_v, out_v,
                 bias_v, sem):
    c = lax.axis_index("c")
    s = lax.axis_index("s")
    wid = s * NC + c
    base = wid * BPW

    pltpu.sync_copy(bias_hbm, bias_v)
    pltpu.sync_copy(idx_hbm.at[pl.ds(wid * CHUNK, CHUNK)], idx_v)

    pltpu.async_copy(table_hbm.at[idx_v], g_v, sem).wait()

    bias16 = bias_v[...]

    def reduce_rows(rg, carry):
        r0 = rg * 16
        acc = g_v[pl.ds(r0, 16)]
        for f in range(1, F):
            acc = acc + g_v[pl.ds(f * BPW + r0, 16)]
        out_v[pl.ds(r0, 16)] = acc + bias16
        return carry

    lax.fori_loop(0, RG, reduce_rows, 0)

    pltpu.sync_copy(out_v, out_hbm.at[pl.ds(base, BPW)])


@jax.jit
def kernel(x, table, bias):
    xt = x.T.reshape(-1)
    tf = table.reshape(-1)
    b16 = jnp.broadcast_to(bias, (16,)).astype(jnp.float32)

    mesh = plsc.VectorSubcoreMesh(core_axis_name="c", subcore_axis_name="s")
    build_idx = pl.kernel(
        _idx_body,
        out_type=jax.ShapeDtypeStruct((B * F,), jnp.int32),
        mesh=mesh,
        scratch_types=[
            pltpu.VMEM((CHUNK,), jnp.int32),
            pltpu.VMEM((CHUNK,), jnp.int32),
            pltpu.SemaphoreType.DMA,
        ],
    )
    gather_sum = pl.kernel(
        _gather_body,
        out_type=jax.ShapeDtypeStruct((B,), jnp.float32),
        mesh=mesh,
        scratch_types=[
            pltpu.VMEM((CHUNK,), jnp.int32),
            pltpu.VMEM((CHUNK,), jnp.float32),
            pltpu.VMEM((BPW,), jnp.float32),
            pltpu.VMEM((16,), jnp.float32),
            pltpu.SemaphoreType.DMA,
        ],
    )
    idx_all = build_idx(xt)
    out = gather_sum(idx_all, tf, b16)
    return out.reshape(B, 1)
